# baseline (speedup 1.0000x reference)
import functools

import jax
import jax.numpy as jnp
from jax import lax
from jax.experimental import pallas as pl
from jax.experimental.pallas import tpu as pltpu
from jax.experimental.pallas import tpu_sc as plsc

_INFO = plsc.get_sparse_core_info()
_NC = _INFO.num_cores
_NS = _INFO.num_subcores
_NW = _NC * _NS
_L = _INFO.num_lanes


def _build(batch, seq, hidden):
    spw = seq // _NW
    q = 8
    ng = spw // q
    grows = batch * q
    rpw = batch * spw
    ring = 3
    nv = hidden // _L
    kunroll = 16
    mesh = plsc.VectorSubcoreMesh(core_axis_name="c", subcore_axis_name="s")

    def body(tok_hbm, ids_hbm, pos_hbm, out_hbm,
             idx_v, pos_buf, tok_buf, pos_sem, gad_sem, out_sem):
        cid = lax.axis_index("c")
        sid = lax.axis_index("s")
        wid = sid * _NC + cid
        s_base = wid * spw

        pltpu.sync_copy(ids_hbm.at[pl.ds(wid * rpw, rpw)], idx_v)

        def _gather_desc(j):
            ts = lax.rem(j, ring)
            return pltpu.make_async_copy(
                tok_hbm.at[idx_v.at[pl.ds(j * grows, grows)]],
                tok_buf.at[ts], gad_sem.at[ts])

        def _pos_desc(j):
            ps = lax.rem(j, ring)
            return pltpu.make_async_copy(
                pos_hbm.at[pl.ds(s_base + j * q, q)],
                pos_buf.at[ps], pos_sem.at[ps])

        def _scatter_descs(j):
            ts = lax.rem(j, ring)
            return [
                pltpu.make_async_copy(
                    tok_buf.at[ts, pl.ds(b * q, q)],
                    out_hbm.at[pl.ds(b * seq + s_base + j * q, q)],
                    out_sem.at[ts * batch + b])
                for b in range(batch)
            ]

        def gather(j):
            _gather_desc(j).start()

        def pos_load(j):
            _pos_desc(j).start()

        def scatter(j):
            for d in _scatter_descs(j):
                d.start()

        gather(0)
        pos_load(0)
        pos_load(1)

        def group(j, _):
            ts = lax.rem(j, ring)

            @pl.when(j + 1 < ng)
            def _():
                @pl.when(j >= 2)
                def _():
                    for d in _scatter_descs(j - 2):
                        d.wait()
                gather(j + 1)

                @pl.when(j + 2 < ng)
                def _():
                    pos_load(j + 2)

            _gather_desc(j).wait()
            _pos_desc(j).wait()

            def row(r, _):
                for k in range(nv):
                    sl = pl.ds(k * _L, _L)
                    p = pos_buf[ts, r, sl]
                    for b in range(batch):
                        tok_buf[ts, b * q + r, sl] = (
                            tok_buf[ts, b * q + r, sl] + p)
                return 0

            lax.fori_loop(0, q, row, 0)
            scatter(j)
            return 0

        lax.fori_loop(0, ng, group, 0)
        for j in (ng - 2, ng - 1):
            for d in _scatter_descs(j):
                d.wait()

    return pl.kernel(
        body,
        out_type=jax.ShapeDtypeStruct((batch * seq, hidden), jnp.float32),
        mesh=mesh,
        scratch_types=[
            pltpu.VMEM((rpw,), jnp.int32),
            pltpu.VMEM((ring, q, hidden), jnp.float32),
            pltpu.VMEM((ring, grows, hidden), jnp.float32),
            pltpu.SemaphoreType.DMA((ring,)),
            pltpu.SemaphoreType.DMA((ring,)),
            pltpu.SemaphoreType.DMA((ring * batch,)),
        ],
    )


def _tc_build(nb, seq, hidden, rb):
    nsteps = nb * seq // rb
    spb = seq // rb

    def tc_body(ids_ref, *refs):
        tabs = refs[:rb]
        pos_ref = refs[rb]
        out_ref = refs[rb + 1]
        for i in range(rb):
            out_ref[i, :] = tabs[i][0, 0, :] + pos_ref[i, :]

    in_specs = [
        pl.BlockSpec((1, 1, hidden),
                     (lambda s, ids, k=k: (ids[s * rb + k], 0, 0)))
        for k in range(rb)
    ] + [pl.BlockSpec((rb, hidden), lambda s, ids: (s % spb, 0))]
    grid_spec = pltpu.PrefetchScalarGridSpec(
        num_scalar_prefetch=1,
        grid=(nsteps,),
        in_specs=in_specs,
        out_specs=pl.BlockSpec((rb, hidden), lambda s, ids: (s, 0)),
    )
    return pl.pallas_call(
        tc_body,
        grid_spec=grid_spec,
        out_shape=jax.ShapeDtypeStruct((nb * seq, hidden), jnp.float32),
    )


@jax.jit
def kernel(input_ids, token_table, pos_table):
    b, seq = input_ids.shape
    hidden = token_table.shape[1]
    spw = seq // _NW
    q = 8
    ng = spw // q
    nb_tc = 1
    nb_sc = b - nb_tc
    ids = input_ids.astype(jnp.int32)
    sc_ids = (ids[:nb_sc]
              .reshape(nb_sc, _NW, ng, q)
              .transpose(1, 2, 0, 3)
              .reshape(-1))
    sc_out = _build(nb_sc, seq, hidden)(token_table, sc_ids, pos_table)
    tc_ids = ids[nb_sc:].reshape(-1)
    tok3 = token_table.reshape(token_table.shape[0], 1, hidden)
    tc_out = _tc_build(nb_tc, seq, hidden, 16)(
        tc_ids, *([tok3] * 16), pos_table)
    return jnp.concatenate(
        [sc_out.reshape(nb_sc, seq, hidden),
         tc_out.reshape(nb_tc, seq, hidden)], axis=0)


# speedup vs baseline: 8.1090x; 8.1090x over previous
import functools

import jax
import jax.numpy as jnp
from jax import lax
from jax.experimental import pallas as pl
from jax.experimental.pallas import tpu as pltpu
from jax.experimental.pallas import tpu_sc as plsc

_INFO = plsc.get_sparse_core_info()
_NC = _INFO.num_cores
_NS = _INFO.num_subcores
_NW = _NC * _NS
_L = _INFO.num_lanes


def _build(batch, seq, hidden):
    spw = seq // _NW
    q = 8
    ng = spw // q
    grows = batch * q
    rpw = batch * spw
    ring = 3
    nv = hidden // _L
    kunroll = 16
    mesh = plsc.VectorSubcoreMesh(core_axis_name="c", subcore_axis_name="s")

    def body(tok_hbm, ids_hbm, pos_hbm, out_hbm,
             idx_v, pos_buf, tok_buf, idx_sem, pos_sem, gad_sem, out_sem):
        cid = lax.axis_index("c")
        sid = lax.axis_index("s")
        wid = sid * _NC + cid
        s_base = wid * spw

        idx_d = pltpu.make_async_copy(
            ids_hbm.at[pl.ds(wid * rpw, rpw)], idx_v, idx_sem)
        idx_d.start()

        def _gather_desc(j):
            ts = lax.rem(j, ring)
            return pltpu.make_async_copy(
                tok_hbm.at[idx_v.at[pl.ds(j * grows, grows)]],
                tok_buf.at[ts], gad_sem.at[ts])

        def _pos_desc(j):
            ps = lax.rem(j, ring)
            return pltpu.make_async_copy(
                pos_hbm.at[pl.ds(s_base + j * q, q)],
                pos_buf.at[ps], pos_sem.at[ps])

        def _scatter_descs(j):
            ts = lax.rem(j, ring)
            return [
                pltpu.make_async_copy(
                    tok_buf.at[ts, pl.ds(b * q, q)],
                    out_hbm.at[pl.ds(b * seq + s_base + j * q, q)],
                    out_sem.at[ts * batch + b])
                for b in range(batch)
            ]

        def gather(j):
            _gather_desc(j).start()

        def pos_load(j):
            _pos_desc(j).start()

        def scatter(j):
            for d in _scatter_descs(j):
                d.start()

        pos_load(0)
        pos_load(1)
        idx_d.wait()
        gather(0)

        def group(j, _):
            ts = lax.rem(j, ring)

            @pl.when(j + 1 < ng)
            def _():
                @pl.when(j >= 2)
                def _():
                    for d in _scatter_descs(j - 2):
                        d.wait()
                gather(j + 1)

                @pl.when(j + 2 < ng)
                def _():
                    pos_load(j + 2)

            _gather_desc(j).wait()
            _pos_desc(j).wait()

            def row(r, _):
                for k in range(nv):
                    sl = pl.ds(k * _L, _L)
                    p = pos_buf[ts, r, sl]
                    for b in range(batch):
                        tok_buf[ts, b * q + r, sl] = (
                            tok_buf[ts, b * q + r, sl] + p)
                return 0

            lax.fori_loop(0, q, row, 0)
            scatter(j)
            return 0

        lax.fori_loop(0, ng, group, 0)
        for j in (ng - 2, ng - 1):
            for d in _scatter_descs(j):
                d.wait()

    return pl.kernel(
        body,
        out_type=jax.ShapeDtypeStruct((batch * seq, hidden), jnp.float32),
        mesh=mesh,
        scratch_types=[
            pltpu.VMEM((rpw,), jnp.int32),
            pltpu.VMEM((ring, q, hidden), jnp.float32),
            pltpu.VMEM((ring, grows, hidden), jnp.float32),
            pltpu.SemaphoreType.DMA,
            pltpu.SemaphoreType.DMA((ring,)),
            pltpu.SemaphoreType.DMA((ring,)),
            pltpu.SemaphoreType.DMA((ring * batch,)),
        ],
    )


@jax.jit
def kernel(input_ids, token_table, pos_table):
    b, seq = input_ids.shape
    hidden = token_table.shape[1]
    spw = seq // _NW
    q = 8
    ng = spw // q
    ids = (input_ids.astype(jnp.int32)
           .reshape(b, _NW, ng, q)
           .transpose(1, 2, 0, 3)
           .reshape(-1))
    out = _build(b, seq, hidden)(token_table, ids, pos_table)
    return out.reshape(b, seq, hidden)
